# baseline (speedup 1.0000x reference)
import functools

import jax
import jax.numpy as jnp
from jax import lax
from jax.experimental import pallas as pl
from jax.experimental.pallas import tpu as pltpu
from jax.experimental.pallas import tpu_sc as plsc

B = 16384
D = 16
V = 1000000
NC = 2
NS = 16
NW = NC * NS
RPW = B // NW
HALF = RPW // 2


def _sc_body(i_hbm, j_hbm, tbl, out, pts_i, pts_j, gi, gj, st_i, st_j,
             q_v, sem):
    c = lax.axis_index("c")
    s = lax.axis_index("s")
    wid = s * NC + c

    pltpu.sync_copy(i_hbm.at[pl.ds(wid * RPW, RPW)], pts_i)
    pltpu.sync_copy(j_hbm.at[pl.ds(wid * RPW, RPW)], pts_j)

    lane = lax.iota(jnp.int32, 16)

    def shift_blk(b, _):
        r = b * 16 + lane
        plsc.store_scatter(gi, [r], plsc.load_gather(pts_i, [r]) >> 3)
        plsc.store_scatter(gj, [r], plsc.load_gather(pts_j, [r]) >> 3)
        return _

    lax.fori_loop(0, RPW // 16, shift_blk, None)

    for half in range(2):
        copies = []
        for g, st in ((gi, st_i), (gj, st_j)):
            for sl in range(HALF // 128):
                copies.append(pltpu.async_copy(
                    tbl.at[g.at[pl.ds(half * HALF + sl * 128, 128)]],
                    st.at[pl.ds(sl * 128, 128)], sem))
        for cp in copies:
            cp.wait()

        def blk_body(b, _, half=half):
            rloc = b * 16 + lane
            pglob = half * HALF + rloc
            pv_i = plsc.load_gather(pts_i, [pglob])
            pv_j = plsc.load_gather(pts_j, [pglob])
            ci = (pv_i & 7) * 16
            cj = (pv_j & 7) * 16
            d = jnp.zeros((16,), jnp.float32)
            for f in range(D):
                zi = plsc.load_gather(st_i, [rloc, ci + f])
                zj = plsc.load_gather(st_j, [rloc, cj + f])
                df = zi - zj + 1e-6
                d = d + df * df
            q = 1.0 / (1.0 + d)
            plsc.store_scatter(q_v, [pglob], q)
            return _

        lax.fori_loop(0, HALF // 16, blk_body, None)

    pltpu.sync_copy(q_v, out.at[pl.ds(wid * RPW, RPW)])


@jax.jit
def _sc_qij(i, j, tbl2):
    mesh = plsc.VectorSubcoreMesh(core_axis_name="c", subcore_axis_name="s")
    f = pl.kernel(
        _sc_body,
        mesh=mesh,
        compiler_params=pltpu.CompilerParams(
            needs_layout_passes=False, use_tc_tiling_on_sc=False),
        out_type=jax.ShapeDtypeStruct((B,), jnp.float32),
        scratch_types=[
            pltpu.VMEM((RPW,), jnp.int32),
            pltpu.VMEM((RPW,), jnp.int32),
            pltpu.VMEM((RPW,), jnp.int32),
            pltpu.VMEM((RPW,), jnp.int32),
            pltpu.VMEM((HALF, 128), jnp.float32),
            pltpu.VMEM((HALF, 128), jnp.float32),
            pltpu.VMEM((RPW,), jnp.float32),
            pltpu.SemaphoreType.DMA,
        ],
    )
    return f(i, j, tbl2)


TBLK = 8192


def _tc_transpose_body(in_ref, out_ref):
    ident = jnp.eye(D, dtype=jnp.float32)
    out_ref[...] = jax.lax.dot_general(
        in_ref[...], ident, (((0,), (0,)), ((), ())),
        preferred_element_type=jnp.float32)


@jax.jit
def _tc_relayout(tbl_t):
    grid = (V + TBLK - 1) // TBLK
    return pl.pallas_call(
        _tc_transpose_body,
        grid=(grid,),
        in_specs=[pl.BlockSpec((D, TBLK), lambda b: (0, b))],
        out_specs=pl.BlockSpec((TBLK, D), lambda b: (b, 0)),
        out_shape=jax.ShapeDtypeStruct((V, D), jnp.float32),
    )(tbl_t)


def _tc_body(p_ref, q_ref, out_ref):
    q = q_ref[...]
    p = p_ref[...]
    s = jnp.sum(q)
    log_q = jnp.log(q / s + 1e-10)
    p_log_p = jnp.where(p > 0, p * jnp.log(jnp.where(p > 0, p, 1.0)), 0.0)
    out_ref[...] = jnp.full((1, 1), jnp.sum(p_log_p - p * log_q), jnp.float32)


@jax.jit
def _tc_loss(p2d, q2d):
    return pl.pallas_call(
        _tc_body,
        out_shape=jax.ShapeDtypeStruct((1, 1), jnp.float32),
    )(p2d, q2d)


def kernel(pij, i, j, logits_weight):
    tbl_rm = _tc_relayout(logits_weight.T)
    q = _sc_qij(i, j, tbl_rm.reshape(V // 8, 8 * D))
    loss = _tc_loss(pij.reshape(128, 128), q.reshape(128, 128))
    return loss[0, 0]


# speedup vs baseline: 2.2063x; 2.2063x over previous
import jax
import jax.numpy as jnp
from jax import lax
from jax.experimental import pallas as pl
from jax.experimental.pallas import tpu as pltpu
from jax.experimental.pallas import tpu_sc as plsc

B = 16384
D = 16
V = 1000000
NC = 2
NS = 16
NW = NC * NS
RANGE = 32768
WIN = 2048
NWIN = 16
BCAP = 256
NBKT = 17
SELCAP = 3072
QCH = 4096
ROWS_PW = NBKT * BCAP
SROWS = 40
TAIL0 = 999424
TAILN = 512
TAIL2 = V - 64
CLAMP0 = TAIL0 - WIN
ZPAD = NW * 128
ZROWS = 2 * B + ZPAD


def _a_body(i_hbm, j_hbm, tbl3, tail64, val_out, slot_out,
            qv, sel_pt, sel_slot, bq, bs, win_a, stv_a,
            srows, tbuf, sem_in, sem_out):
    c = lax.axis_index("c")
    s = lax.axis_index("s")
    wid = s * NC + c
    lane = lax.iota(jnp.int32, 16)

    pltpu.sync_copy(tail64, tbuf)

    off = jnp.int32(0)
    for qi, qsrc in enumerate((i_hbm, j_hbm)):
        for ch in range(B // QCH):
            pltpu.sync_copy(qsrc.at[pl.ds(ch * QCH, QCH)], qv)
            sbase = qi * B + ch * QCH

            def scan(v, o, sbase=sbase):
                i16 = v * 16 + lane
                pts = plsc.load_gather(qv, [i16])
                m = (pts >> 15) == wid
                plsc.store_compressed(sel_pt.at[pl.ds(o, 16)], pts, mask=m)
                plsc.store_compressed(
                    sel_slot.at[pl.ds(o, 16)], i16 + sbase, mask=m)
                return o + plsc.all_reduce_population_count(m)[0]

            off = lax.fori_loop(0, QCH // 16, scan, off)

    def prefill(v, _):
        r = v * 16 + lane
        plsc.store_scatter(bq, [r], jnp.zeros((16,), jnp.int32))
        plsc.store_scatter(bs, [r], 2 * B + wid * 128 + (r & 127))
        return _

    lax.fori_loop(0, ROWS_PW // 16, prefill, None)

    ntrip = (off + 15) >> 4
    for b in range(NBKT):
        def binb(v, ob, b=b):
            i16 = v * 16 + lane
            pts = plsc.load_gather(sel_pt, [i16])
            slots = plsc.load_gather(sel_slot, [i16])
            bkt = jnp.where(pts >= TAIL0, NBKT - 1,
                            (pts - wid * RANGE) >> 11)
            m = (bkt == b) & (i16 < off)
            plsc.store_compressed(bq.at[pl.ds(b * BCAP + ob, 16)], pts, mask=m)
            plsc.store_compressed(bs.at[pl.ds(b * BCAP + ob, 16)], slots, mask=m)
            return ob + plsc.all_reduce_population_count(m)[0]

        lax.fori_loop(0, ntrip, binb, jnp.int32(0))

    def wstart(win):
        if win == NBKT - 1:
            return jnp.int32(TAIL0), TAILN
        raw = wid * RANGE + win * WIN
        return pl.multiple_of(jnp.minimum(raw, CLAMP0), 128), WIN

    def fetch(win):
        buf = win_a
        st, ln = wstart(win)
        cps = []
        for fh in range(2):
            cps.append(pltpu.async_copy(
                tbl3.at[fh, :, pl.ds(st, ln)],
                buf.at[pl.ds(fh * 8, 8), pl.ds(0, ln)], sem_in))
        return cps

    pending_v = []
    for win in range(NBKT):
        for cp in fetch(win):
            cp.wait()
        buf = win_a
        st, _ = wstart(win)
        stv = stv_a
        if len(pending_v) >= 1:
            pending_v.pop(0).wait()

        def ext(e, _, win=win, buf=buf, stv=stv, st=st):
            k = win * BCAP + e * 16 + lane
            pt = plsc.load_gather(bq, [k])
            sl = plsc.load_gather(bs, [k])
            ploc = jnp.clip(pt - st, 0, TAILN - 1 if win == NBKT - 1
                            else WIN - 1)
            if win == NBKT - 1:
                in_tail2 = pt >= TAIL2
                trow = jnp.clip(pt - TAIL2, 0, 63)
            plsc.store_scatter(
                srows,
                [jnp.full((16,), win * 2, jnp.int32) + (e >> 3),
                 (e & 7) * 16 + lane], sl)
            for f in range(D):
                fv = jnp.full((16,), f, jnp.int32)
                vals = plsc.load_gather(buf, [fv, ploc])
                if win == NBKT - 1:
                    tvals = plsc.load_gather(tbuf, [trow, fv])
                    vals = jnp.where(in_tail2, tvals, vals)
                plsc.store_scatter(
                    stv, [e * 16 + lane, fv], vals)
            return _

        lax.fori_loop(0, BCAP // 16, ext, None)
        dst_row = pl.multiple_of(wid * ROWS_PW + win * BCAP, 128)
        pending_v.append(pltpu.async_copy(
            stv, val_out.at[pl.ds(dst_row, BCAP), :], sem_out))
    for cp in pending_v:
        cp.wait()

    pltpu.sync_copy(srows, slot_out.at[pl.ds(wid * SROWS, SROWS), :])


@jax.jit
def _a_stream(i, j, tbl3, tail64):
    mesh = plsc.VectorSubcoreMesh(core_axis_name="c", subcore_axis_name="s")
    f = pl.kernel(
        _a_body,
        mesh=mesh,
        compiler_params=pltpu.CompilerParams(
            needs_layout_passes=False, use_tc_tiling_on_sc=True),
        out_type=(
            jax.ShapeDtypeStruct((NW * ROWS_PW, D), jnp.float32),
            jax.ShapeDtypeStruct((NW * SROWS, 128), jnp.int32),
        ),
        scratch_types=[
            pltpu.VMEM((QCH,), jnp.int32),
            pltpu.VMEM((SELCAP,), jnp.int32),
            pltpu.VMEM((SELCAP,), jnp.int32),
            pltpu.VMEM((ROWS_PW,), jnp.int32),
            pltpu.VMEM((ROWS_PW,), jnp.int32),
            pltpu.VMEM((D, WIN), jnp.float32),
            pltpu.VMEM((BCAP, D), jnp.float32),
            pltpu.VMEM((SROWS, 128), jnp.int32),
            pltpu.VMEM((64, D), jnp.float32),
            pltpu.SemaphoreType.DMA,
            pltpu.SemaphoreType.DMA,
        ],
    )
    return f(i, j, tbl3, tail64)


def _b_body(val_all, slot_all, zbuf, vstage, sstage, sem):
    c = lax.axis_index("c")
    s = lax.axis_index("s")
    wid = s * NC + c
    pltpu.sync_copy(val_all.at[pl.ds(wid * ROWS_PW, ROWS_PW)], vstage)
    pltpu.sync_copy(slot_all.at[pl.ds(wid * SROWS, SROWS)], sstage)
    cps = []
    for r in range(2 * NBKT):
        cps.append(pltpu.async_copy(
            vstage.at[pl.ds(r * 128, 128)], zbuf.at[sstage.at[r]], sem))
    for cp in cps:
        cp.wait()


@jax.jit
def _b_scatter(val_all, slot_all):
    mesh = plsc.VectorSubcoreMesh(core_axis_name="c", subcore_axis_name="s")
    f = pl.kernel(
        _b_body,
        mesh=mesh,
        compiler_params=pltpu.CompilerParams(
            needs_layout_passes=False, use_tc_tiling_on_sc=False),
        out_type=jax.ShapeDtypeStruct((ZROWS, D), jnp.float32),
        scratch_types=[
            pltpu.VMEM((ROWS_PW, D), jnp.float32),
            pltpu.VMEM((SROWS, 128), jnp.int32),
            pltpu.SemaphoreType.DMA,
        ],
    )
    return f(val_all, slot_all)


def _tc_body(p_ref, zi_ref, zj_ref, out_ref):
    zi = zi_ref[...]
    zj = zj_ref[...]
    p = p_ref[...]
    df = zi - zj + 1e-6
    d = jnp.sum(df * df, axis=1).reshape(128, 128)
    q = 1.0 / (1.0 + d)
    sq = jnp.sum(q)
    log_q = jnp.log(q / sq + 1e-10)
    p_log_p = jnp.where(p > 0, p * jnp.log(jnp.where(p > 0, p, 1.0)), 0.0)
    out_ref[...] = jnp.full((1, 1), jnp.sum(p_log_p - p * log_q), jnp.float32)


@jax.jit
def _tc_loss(p2d, zi, zj):
    return pl.pallas_call(
        _tc_body,
        out_shape=jax.ShapeDtypeStruct((1, 1), jnp.float32),
    )(p2d, zi, zj)


def kernel(pij, i, j, logits_weight):
    tbl3 = logits_weight.T.reshape(2, 8, V)
    val_all, slot_all = _a_stream(i, j, tbl3, logits_weight[TAIL2:])
    zbuf = _b_scatter(val_all, slot_all)
    loss = _tc_loss(pij.reshape(128, 128), zbuf[:B], zbuf[B:2 * B])
    return loss[0, 0]
